# baseline (speedup 1.0000x reference)
import functools
import jax
import jax.numpy as jnp
from jax import lax
from jax.experimental import pallas as pl
from jax.experimental.pallas import tpu as pltpu
from jax.experimental.pallas import tpu_sc as plsc

D_MODEL = 64
SCALE = 8.0

B_TOTAL = 4096 * 200
NUM_WORKERS = 32
B_PER_W = B_TOTAL // NUM_WORKERS
CHUNK = 128
NCHUNK = B_PER_W // CHUNK
LANES = 16
VECS_PER_ROW = D_MODEL // LANES

_mesh = plsc.VectorSubcoreMesh(core_axis_name="c", subcore_axis_name="s")


@functools.partial(
    pl.kernel,
    mesh=_mesh,
    out_type=jax.ShapeDtypeStruct((B_TOTAL, D_MODEL), jnp.float32),
    scratch_types=[
        pltpu.VMEM((NCHUNK, CHUNK), jnp.int32),
        pltpu.VMEM((CHUNK, D_MODEL), jnp.float32),
        pltpu.SemaphoreType.DMA,
    ],
    compiler_params=pltpu.CompilerParams(use_tc_tiling_on_sc=False),
)
def _embed_sc(x_hbm, table_hbm, out_hbm, idx_v, rows_v, sem):
    wid = lax.axis_index("s") * 2 + lax.axis_index("c")
    pltpu.sync_copy(x_hbm.at[pl.ds(wid * NCHUNK, NCHUNK)], idx_v)

    def chunk_body(j, _):
        pltpu.async_copy(table_hbm.at[idx_v.at[j]], rows_v, sem).wait()

        def scale_body(r, _):
            for c in range(VECS_PER_ROW):
                sl = pl.ds(c * LANES, LANES)
                rows_v[r, sl] = rows_v[r, sl] * SCALE
            return 0

        lax.fori_loop(0, CHUNK, scale_body, 0)
        pltpu.sync_copy(rows_v, out_hbm.at[pl.ds(wid * B_PER_W + j * CHUNK, CHUNK)])
        return 0

    lax.fori_loop(0, NCHUNK, chunk_body, 0)


def kernel(x, table):
    xf = x.reshape(NUM_WORKERS * NCHUNK, CHUNK)
    out = _embed_sc(xf, table)
    return out.reshape(4096, 200, D_MODEL)


# speedup vs baseline: 1.2094x; 1.2094x over previous
import functools
import jax
import jax.numpy as jnp
from jax import lax
from jax.experimental import pallas as pl
from jax.experimental.pallas import tpu as pltpu
from jax.experimental.pallas import tpu_sc as plsc

D_MODEL = 64
SCALE = 8.0

B_TOTAL = 4096 * 200
NUM_WORKERS = 32
B_PER_W = B_TOTAL // NUM_WORKERS
CHUNK = 128
NCHUNK = B_PER_W // CHUNK
LANES = 16
VECS_PER_ROW = D_MODEL // LANES
NBUF = 4
NGROUP = NCHUNK // NBUF

_mesh = plsc.VectorSubcoreMesh(core_axis_name="c", subcore_axis_name="s")


@functools.partial(
    pl.kernel,
    mesh=_mesh,
    out_type=jax.ShapeDtypeStruct((B_TOTAL, D_MODEL), jnp.float32),
    scratch_types=[
        pltpu.VMEM((NCHUNK, CHUNK), jnp.int32),
        pltpu.VMEM((NBUF, CHUNK, D_MODEL), jnp.float32),
        pltpu.VMEM((NBUF, CHUNK, D_MODEL), jnp.float32),
        pltpu.SemaphoreType.DMA((NBUF,)),
        pltpu.SemaphoreType.DMA((NBUF,)),
    ],
    compiler_params=pltpu.CompilerParams(use_tc_tiling_on_sc=False),
)
def _embed_sc(x_hbm, table_hbm, out_hbm, idx_v, rows_v, obuf_v, gsem, osem):
    wid = lax.axis_index("s") * 2 + lax.axis_index("c")
    out_base = wid * B_PER_W
    pltpu.sync_copy(x_hbm.at[pl.ds(wid * NCHUNK, NCHUNK)], idx_v)

    for b in range(NBUF):
        pltpu.async_copy(table_hbm.at[idx_v.at[b]], rows_v.at[b], gsem.at[b])

    def group_body(g, _):
        j0 = g * NBUF
        for b in range(NBUF):
            j = j0 + b
            pltpu.make_async_copy(
                table_hbm.at[idx_v.at[j]], rows_v.at[b], gsem.at[b]
            ).wait()

            @pl.when(j >= NBUF)
            def _():
                pltpu.make_async_copy(
                    obuf_v.at[b],
                    out_hbm.at[pl.ds(out_base + (j - NBUF) * CHUNK, CHUNK)],
                    osem.at[b],
                ).wait()

            def scale_body(r, _):
                for c in range(VECS_PER_ROW):
                    sl = pl.ds(c * LANES, LANES)
                    obuf_v[b, r, sl] = rows_v[b, r, sl] * SCALE
                return 0

            lax.fori_loop(0, CHUNK, scale_body, 0)

            @pl.when(j + NBUF < NCHUNK)
            def _():
                pltpu.async_copy(
                    table_hbm.at[idx_v.at[j + NBUF]], rows_v.at[b], gsem.at[b]
                )

            pltpu.async_copy(
                obuf_v.at[b],
                out_hbm.at[pl.ds(out_base + j * CHUNK, CHUNK)],
                osem.at[b],
            )
        return 0

    lax.fori_loop(0, NGROUP, group_body, 0)

    for b in range(NBUF):
        j = NCHUNK - NBUF + b
        pltpu.make_async_copy(
            obuf_v.at[b],
            out_hbm.at[pl.ds(out_base + j * CHUNK, CHUNK)],
            osem.at[b],
        ).wait()


def kernel(x, table):
    xf = x.reshape(NUM_WORKERS * NCHUNK, CHUNK)
    out = _embed_sc(xf, table)
    return out.reshape(4096, 200, D_MODEL)
